# baseline (speedup 1.0000x reference)
import functools

import numpy as np
import jax
import jax.numpy as jnp
from jax import lax
from jax.experimental import pallas as pl
from jax.experimental.pallas import tpu as pltpu
from jax.experimental.pallas import tpu_sc as plsc

NN = 10000
EE = 320000
FD = 128
NB = 64
NPAD = 10240
DEAD = NN
NW = 32
EPW = EE // NW
CH = 80
NCHE = EPW // CH
RPT = NPAD // NW
RCH = 80
NRCH = RPT // RCH
NSUB = 16
RPS = NPAD // NSUB
K1, K2, K3 = 5000, 2500, 1250
NEG = -1.0e30
SENT = -3.0e38

_mesh = plsc.VectorSubcoreMesh(core_axis_name="c", subcore_axis_name="s")
_SC_PARAMS = pltpu.CompilerParams(needs_layout_passes=False)
_ONEHOT = [np.eye(16, dtype=np.float32)[j] for j in range(16)]


def _wid():
    return lax.axis_index("s") * 2 + lax.axis_index("c")


def _f32(shape):
    return jax.ShapeDtypeStruct(shape, jnp.float32)


def _i32(shape):
    return jax.ShapeDtypeStruct(shape, jnp.int32)


@functools.partial(
    pl.kernel,
    out_type=(_i32((EE,)), _i32((EE,)), _f32((NW, NPAD))),
    mesh=_mesh,
    compiler_params=_SC_PARAMS,
    scratch_types=[
        pltpu.VMEM((NPAD,), jnp.float32),
        pltpu.VMEM((EPW,), jnp.int32),
        pltpu.VMEM((EPW,), jnp.int32),
        pltpu.VMEM((EPW,), jnp.int32),
        pltpu.VMEM((EPW,), jnp.int32),
        pltpu.VMEM((NPAD,), jnp.float32),
    ],
)
def sc_prep(src_h, dstv_h, m_h, srcv_o, dstv_o, degp_o,
            m_v, src_v, dst_v, sout_v, dout_v, deg_v):
    w = _wid()
    base = w * EPW
    pltpu.sync_copy(m_h, m_v)
    pltpu.sync_copy(src_h.at[pl.ds(base, EPW)], src_v)
    pltpu.sync_copy(dstv_h.at[pl.ds(base, EPW)], dst_v)

    def zero(i, _):
        deg_v[pl.ds(i * 16, 16)] = jnp.zeros((16,), jnp.float32)
        return 0

    lax.fori_loop(0, NPAD // 16, zero, 0)
    ones16 = jnp.ones((16,), jnp.float32)
    dead16 = jnp.full((16,), DEAD, jnp.int32)

    def step(i, _):
        for u in range(5):
            sl = pl.ds((i * 5 + u) * 16, 16)
            si = src_v[sl]
            di = dst_v[sl]
            ms = plsc.load_gather(m_v, [si])
            md = plsc.load_gather(m_v, [di])
            ok = (ms > 0.0) & (md > 0.0)
            dn = jnp.where(ok, di, dead16)
            sn = jnp.where(ok, si, dead16)
            dout_v[sl] = dn
            sout_v[sl] = sn
            plsc.addupdate_scatter(deg_v, [dn], ones16)
        return 0

    lax.fori_loop(0, EPW // 80, step, 0)
    pltpu.sync_copy(sout_v, srcv_o.at[pl.ds(base, EPW)])
    pltpu.sync_copy(dout_v, dstv_o.at[pl.ds(base, EPW)])
    pltpu.sync_copy(deg_v, degp_o.at[w])


@functools.partial(
    pl.kernel,
    out_type=_f32((2, NPAD, FD)),
    mesh=_mesh,
    compiler_params=_SC_PARAMS,
    scratch_types=[
        pltpu.VMEM((CH,), jnp.int32),
        pltpu.VMEM((CH,), jnp.int32),
        pltpu.VMEM((CH,), jnp.int32),
        pltpu.VMEM((CH,), jnp.int32),
        pltpu.VMEM((CH, FD), jnp.float32),
        pltpu.VMEM((CH, FD), jnp.float32),
        pltpu.VMEM_SHARED((NPAD, FD), jnp.float32),
        pltpu.SemaphoreType.DMA,
        pltpu.SemaphoreType.DMA,
        pltpu.SemaphoreType.DMA,
        pltpu.SemaphoreType.DMA,
    ],
)
def sc_conv128(hp_h, src_h, dstv_h, aggp_o, isa, ida, isb, idb, rwa, rwb,
               agg_sh, sia, sib, sga, sgb):
    c = lax.axis_index("c")
    s = lax.axis_index("s")
    w = s * 2 + c
    base = w * EPW

    def zrow(i, _):
        for v in range(FD // 16):
            rwa[i, pl.ds(v * 16, 16)] = jnp.zeros((16,), jnp.float32)
        return 0

    lax.fori_loop(0, CH, zrow, 0)
    for t in range(RPS // CH):
        pltpu.sync_copy(rwa, agg_sh.at[pl.ds(s * RPS + t * CH, CH)])
    plsc.subcore_barrier()

    def issue_idx(i, bs, bd, sem):
        off = base + i * CH
        pltpu.async_copy(src_h.at[pl.ds(off, CH)], bs, sem)
        pltpu.async_copy(dstv_h.at[pl.ds(off, CH)], bd, sem)

    def wait_idx(bs, bd, sem):
        pltpu.make_async_copy(src_h.at[pl.ds(base, CH)], bs, sem).wait()
        pltpu.make_async_copy(dstv_h.at[pl.ds(base, CH)], bd, sem).wait()

    def issue_g(bs, rw, sem):
        pltpu.async_copy(hp_h.at[bs], rw, sem)

    def wait_g(bs, rw, sem):
        pltpu.make_async_copy(hp_h.at[bs], rw, sem).wait()

    issue_idx(0, isa, ida, sia)
    wait_idx(isa, ida, sia)
    issue_g(isa, rwa, sga)
    issue_idx(1, isb, idb, sib)

    def pair(g, _):
        wait_idx(isb, idb, sib)
        wait_g(isa, rwa, sga)
        issue_g(isb, rwb, sgb)
        pltpu.sync_copy(rwa, agg_sh.at[ida], add=True)
        issue_idx(2 * g + 2, isa, ida, sia)
        wait_idx(isa, ida, sia)
        issue_g(isa, rwa, sga)
        wait_g(isb, rwb, sgb)
        pltpu.sync_copy(rwb, agg_sh.at[idb], add=True)

        @pl.when(g < NCHE // 2 - 1)
        def _():
            issue_idx(2 * g + 3, isb, idb, sib)

        return 0

    lax.fori_loop(0, NCHE // 2, pair, 0)
    wait_g(isa, rwa, sga)
    pltpu.sync_copy(rwa, agg_sh.at[ida], add=True)
    plsc.subcore_barrier()
    pltpu.sync_copy(
        agg_sh.at[pl.ds(s * RPS, RPS)], aggp_o.at[c, pl.ds(s * RPS, RPS)]
    )


@functools.partial(
    pl.kernel,
    out_type=_f32((NPAD, FD)),
    mesh=_mesh,
    compiler_params=_SC_PARAMS,
    scratch_types=[
        pltpu.VMEM((RCH, FD), jnp.float32),
        pltpu.VMEM((RPT,), jnp.float32),
    ],
)
def sc_hprep(h_h, dinv_h, hp_o, hb, db):
    w = _wid()
    n0 = w * RPT
    pltpu.sync_copy(dinv_h.at[pl.ds(n0, RPT)], db)
    for ch in range(NRCH):
        r0 = n0 + ch * RCH
        pltpu.sync_copy(h_h.at[pl.ds(r0, RCH)], hb)

        def grp(g, _):
            dvec = db[pl.ds(ch * RCH + g * 16, 16)]
            for j in range(16):
                dv = dvec[j]
                r = g * 16 + j
                for v in range(FD // 16):
                    hb[r, pl.ds(v * 16, 16)] = hb[r, pl.ds(v * 16, 16)] * dv
            return 0

        lax.fori_loop(0, RCH // 16, grp, 0)
        pltpu.sync_copy(hb, hp_o.at[pl.ds(r0, RCH)])


@functools.partial(
    pl.kernel,
    out_type=(_f32((NPAD, FD)), _f32((NPAD,))),
    mesh=_mesh,
    compiler_params=_SC_PARAMS,
    scratch_types=[
        pltpu.VMEM((RCH, FD), jnp.float32),
        pltpu.VMEM((RCH, FD), jnp.float32),
        pltpu.VMEM((RCH, FD), jnp.float32),
        pltpu.VMEM((RCH, FD), jnp.float32),
        pltpu.VMEM((RPT,), jnp.float32),
        pltpu.VMEM((RPT,), jnp.float32),
        pltpu.VMEM((FD,), jnp.float32),
        pltpu.VMEM((FD,), jnp.float32),
        pltpu.VMEM((RPT,), jnp.float32),
    ],
)
def sc_epilogue(aggp_h, h_h, dinv_h, m_h, b_h, wp_h, f_o, pp_o,
                a0, a1, hb, fb, db, mb, bv, wv, ppb):
    w = _wid()
    n0 = w * RPT
    pltpu.sync_copy(dinv_h.at[pl.ds(n0, RPT)], db)
    pltpu.sync_copy(m_h.at[pl.ds(n0, RPT)], mb)
    pltpu.sync_copy(b_h, bv)
    pltpu.sync_copy(wp_h, wv)
    for ch in range(NRCH):
        r0 = n0 + ch * RCH
        pltpu.sync_copy(aggp_h.at[0, pl.ds(r0, RCH)], a0)
        pltpu.sync_copy(aggp_h.at[1, pl.ds(r0, RCH)], a1)
        pltpu.sync_copy(h_h.at[pl.ds(r0, RCH)], hb)

        def grp(g, _):
            gb = ch * RCH + g * 16
            dvec = db[pl.ds(gb, 16)]
            mvec = mb[pl.ds(gb, 16)]
            ivec = lax.iota(jnp.int32, 16)
            ppacc = jnp.zeros((16,), jnp.float32)
            for j in range(16):
                dv = dvec[j]
                mv = mvec[j]
                r = g * 16 + j
                acc = jnp.zeros((16,), jnp.float32)
                for v in range(FD // 16):
                    sl = pl.ds(v * 16, 16)
                    val = (a0[r, sl] + a1[r, sl] + hb[r, sl]) * dv + bv[sl]
                    fv = jnp.maximum(val, 0.0) * mv
                    fb[r, sl] = fv
                    acc = acc + fv * wv[sl]
                ppacc = jnp.where(ivec == j, dv * jnp.sum(acc), ppacc)
            ppb[pl.ds(gb, 16)] = ppacc
            return 0

        lax.fori_loop(0, RCH // 16, grp, 0)
        pltpu.sync_copy(fb, f_o.at[pl.ds(r0, RCH)])
    pltpu.sync_copy(ppb, pp_o.at[pl.ds(n0, RPT)])


@functools.partial(
    pl.kernel,
    out_type=_f32((NW * NPAD,)),
    mesh=_mesh,
    compiler_params=_SC_PARAMS,
    scratch_types=[
        pltpu.VMEM((NPAD,), jnp.float32),
        pltpu.VMEM((EPW,), jnp.int32),
        pltpu.VMEM((EPW,), jnp.int32),
        pltpu.VMEM((NPAD,), jnp.float32),
    ],
)
def sc_sconv(pp_h, src_h, dstv_h, saggp_o, p_v, src_v, dst_v, agg_v):
    w = _wid()
    base = w * EPW
    pltpu.sync_copy(pp_h, p_v)
    pltpu.sync_copy(src_h.at[pl.ds(base, EPW)], src_v)
    pltpu.sync_copy(dstv_h.at[pl.ds(base, EPW)], dst_v)

    def zero(i, _):
        agg_v[pl.ds(i * 16, 16)] = jnp.zeros((16,), jnp.float32)
        return 0

    lax.fori_loop(0, NPAD // 16, zero, 0)

    def step(i, _):
        for u in range(5):
            sl = pl.ds((i * 5 + u) * 16, 16)
            si = src_v[sl]
            di = dst_v[sl]
            vals = plsc.load_gather(p_v, [si])
            plsc.addupdate_scatter(agg_v, [di], vals)
        return 0

    lax.fori_loop(0, EPW // 80, step, 0)
    pltpu.sync_copy(agg_v, saggp_o.at[pl.ds(w * NPAD, NPAD)])


@functools.partial(
    pl.kernel,
    out_type=_f32((NPAD,)),
    mesh=_mesh,
    compiler_params=_SC_PARAMS,
    scratch_types=[
        pltpu.VMEM((NW * RPT,), jnp.float32),
        pltpu.VMEM((RPT,), jnp.float32),
        pltpu.VMEM((RPT,), jnp.float32),
        pltpu.VMEM((RPT,), jnp.float32),
        pltpu.VMEM((16,), jnp.float32),
        pltpu.VMEM((RPT,), jnp.float32),
    ],
)
def sc_sfinish(saggp_h, dinv_h, pp_h, m_h, bp_h, sm_o, sg, db, pb, mb, bpv, smb):
    w = _wid()
    n0 = w * RPT
    for j in range(NW):
        pltpu.sync_copy(saggp_h.at[pl.ds(j * NPAD + n0, RPT)],
                        sg.at[pl.ds(j * RPT, RPT)])
    pltpu.sync_copy(dinv_h.at[pl.ds(n0, RPT)], db)
    pltpu.sync_copy(pp_h.at[pl.ds(n0, RPT)], pb)
    pltpu.sync_copy(m_h.at[pl.ds(n0, RPT)], mb)
    pltpu.sync_copy(bp_h, bpv)

    def grp(g, _):
        sl = pl.ds(g * 16, 16)
        acc = jnp.zeros((16,), jnp.float32)
        for j in range(NW):
            acc = acc + sg[pl.ds(j * RPT + g * 16, 16)]
        sv = db[sl] * acc + db[sl] * pb[sl] + bpv[pl.ds(0, 16)]
        smb[sl] = jnp.where(mb[sl] > 0.0, sv, jnp.full((16,), SENT, jnp.float32))
        return 0

    lax.fori_loop(0, RPT // 16, grp, 0)
    pltpu.sync_copy(smb, sm_o.at[pl.ds(n0, RPT)])


@functools.partial(
    pl.kernel,
    out_type=(
        _f32((NPAD, FD)),
        _f32((NW, NB * FD)),
        _f32((NW, NB * FD)),
        _f32((NW, NB * FD)),
    ),
    mesh=_mesh,
    compiler_params=_SC_PARAMS,
    scratch_types=[
        pltpu.VMEM((RCH, FD), jnp.float32),
        pltpu.VMEM((RCH, FD), jnp.float32),
        pltpu.VMEM((RPT,), jnp.float32),
        pltpu.VMEM((RPT,), jnp.float32),
        pltpu.VMEM((RPT,), jnp.int32),
        pltpu.VMEM((RPT,), jnp.float32),
        pltpu.VMEM((NB * FD,), jnp.float32),
        pltpu.VMEM((NB * FD,), jnp.float32),
        pltpu.VMEM((NB * FD,), jnp.float32),
    ],
)
def sc_xnext(f_h, sm_h, mn_h, batch_h, dn_h, xn_o, sump_o, maxp_o, cntp_o,
             fb, xb, smb, mnb, bb, dnb, sl_, ml_, cl_):
    w = _wid()
    n0 = w * RPT
    pltpu.sync_copy(sm_h.at[pl.ds(n0, RPT)], smb)
    pltpu.sync_copy(mn_h.at[pl.ds(n0, RPT)], mnb)
    pltpu.sync_copy(batch_h.at[pl.ds(n0, RPT)], bb)
    pltpu.sync_copy(dn_h.at[pl.ds(n0, RPT)], dnb)

    def zero(i, _):
        sl_[pl.ds(i * 16, 16)] = jnp.zeros((16,), jnp.float32)
        cl_[pl.ds(i * 16, 16)] = jnp.zeros((16,), jnp.float32)
        ml_[pl.ds(i * 16, 16)] = jnp.full((16,), NEG, jnp.float32)
        return 0

    lax.fori_loop(0, NB * FD // 16, zero, 0)
    for ch in range(NRCH):
        r0 = n0 + ch * RCH
        pltpu.sync_copy(f_h.at[pl.ds(r0, RCH)], fb)

        def grp(g, _):
            gb = ch * RCH + g * 16
            svec = smb[pl.ds(gb, 16)]
            mnvec = mnb[pl.ds(gb, 16)]
            bvec = bb[pl.ds(gb, 16)]
            dnvec = dnb[pl.ds(gb, 16)]
            e = jnp.exp(-2.0 * jnp.abs(svec))
            tco = jnp.sign(svec) * (1.0 - e) / (1.0 + e) * mnvec
            sentv = (mnvec - 1.0) * 1.0e30
            for j in range(16):
                coef = tco[j]
                sent = sentv[j]
                mnr = mnvec[j]
                seg = bvec[j]
                dnr = dnvec[j]
                r = g * 16 + j
                for v in range(FD // 16):
                    sl = pl.ds(v * 16, 16)
                    xv = fb[r, sl] * coef
                    xb[r, sl] = xv * dnr
                    off = pl.ds(seg * FD + v * 16, 16)
                    sl_[off] = sl_[off] + xv
                    ml_[off] = jnp.maximum(ml_[off], xv + sent)
                    cl_[off] = cl_[off] + mnr
            return 0

        lax.fori_loop(0, RCH // 16, grp, 0)
        pltpu.sync_copy(xb, xn_o.at[pl.ds(r0, RCH)])
    pltpu.sync_copy(sl_, sump_o.at[w])
    pltpu.sync_copy(ml_, maxp_o.at[w])
    pltpu.sync_copy(cl_, cntp_o.at[w])


def _tc_mm_body(x_ref, w_ref, o_ref):
    o_ref[...] = jnp.dot(x_ref[...], w_ref[...], preferred_element_type=jnp.float32)


def tc_mm(x, w):
    bm = 1024
    return pl.pallas_call(
        _tc_mm_body,
        grid=(NPAD // bm,),
        in_specs=[
            pl.BlockSpec((bm, FD), lambda i: (i, 0)),
            pl.BlockSpec((FD, FD), lambda i: (0, 0)),
        ],
        out_specs=pl.BlockSpec((bm, FD), lambda i: (i, 0)),
        out_shape=_f32((NPAD, FD)),
    )(x, w)


def _tc_degfinish_body(degp_ref, o_ref):
    deg = jnp.sum(degp_ref[...], axis=0) + 1.0
    o_ref[...] = lax.rsqrt(deg)


def tc_degfinish(degp):
    return pl.pallas_call(
        _tc_degfinish_body,
        out_shape=_f32((NPAD // FD, FD)),
    )(degp.reshape(NW, NPAD // FD, FD))


def _tc_topk_body(k, sm_ref, o_ref):
    sm = sm_ref[...]
    ub = lax.bitcast_convert_type(sm, jnp.uint32)
    sign = (ub >> 31).astype(jnp.uint32)
    u = jnp.where(sign > 0, ~ub, ub | jnp.uint32(0x80000000))
    kf = jnp.float32(k)

    def cnt_ge(t):
        return jnp.sum((u >= t).astype(jnp.float32))

    def vbit(b, t):
        cand = t | (jnp.uint32(1) << (jnp.uint32(31) - b.astype(jnp.uint32)))
        return jnp.where(cnt_ge(cand) >= kf, cand, t)

    t = lax.fori_loop(0, 32, vbit, jnp.uint32(0))
    c_gt = jnp.sum((u > t).astype(jnp.float32))
    r = kf - c_gt
    rows = lax.broadcasted_iota(jnp.int32, sm.shape, 0)
    lanes = lax.broadcasted_iota(jnp.int32, sm.shape, 1)
    idx = rows * FD + lanes
    eq = u == t

    def cnt_le(j):
        return jnp.sum((eq & (idx <= j)).astype(jnp.float32))

    def jbit(b, j):
        cand = j & ~(jnp.int32(1) << (jnp.int32(13) - b))
        return jnp.where(cnt_le(cand) >= r, cand, j)

    j = lax.fori_loop(0, 14, jbit, jnp.int32(16383))
    o_ref[...] = ((u > t) | (eq & (idx <= j))).astype(jnp.float32)


def tc_topk(sm2, k):
    return pl.pallas_call(
        functools.partial(_tc_topk_body, k),
        out_shape=_f32((NPAD // FD, FD)),
    )(sm2)


def _tc_readout_body(sump_ref, maxp_ref, cntp_ref, o_ref):
    s = jnp.sum(sump_ref[...], axis=0)
    c = jnp.sum(cntp_ref[...], axis=0)
    mx = jnp.max(maxp_ref[...], axis=0)
    mx = jnp.where(mx < -1.0e29, 0.0, mx)
    mean = s / jnp.maximum(c, 1.0)
    o_ref[:, :FD] = mx
    o_ref[:, FD:] = mean


def tc_readout(sump, maxp, cntp):
    return pl.pallas_call(
        _tc_readout_body,
        out_shape=_f32((NB, 2 * FD)),
    )(sump.reshape(NW, NB, FD), maxp.reshape(NW, NB, FD), cntp.reshape(NW, NB, FD))


def _tc_mlp_body(x1_ref, x2_ref, x3_ref, w1_ref, b1_ref, w2_ref, b2_ref,
                 w3_ref, b3_ref, o_ref):
    z = x1_ref[...] + x2_ref[...] + x3_ref[...]
    z = jnp.maximum(jnp.dot(z, w1_ref[...], preferred_element_type=jnp.float32)
                    + b1_ref[...], 0.0)
    z = jnp.maximum(jnp.dot(z, w2_ref[...], preferred_element_type=jnp.float32)
                    + b2_ref[...], 0.0)
    lg = jnp.dot(z, w3_ref[...], preferred_element_type=jnp.float32) + b3_ref[...]
    mx = jnp.max(lg, axis=1, keepdims=True)
    ls = jnp.log(jnp.sum(jnp.exp(lg - mx), axis=1, keepdims=True)) + mx
    o_ref[...] = lg - ls


def tc_mlp(x1, x2, x3, w1, b1, w2, b2, w3, b3):
    return pl.pallas_call(
        _tc_mlp_body,
        out_shape=_f32((NB, 10)),
    )(x1, x2, x3, w1, b1.reshape(1, -1), w2, b2.reshape(1, -1),
      w3, b3.reshape(1, -1))


def kernel(x, edge_index, batch, epoch, W1, b1, Wp1, bp1, W2, b2, Wp2, bp2,
           W3, b3, Wp3, bp3, Wl1, bl1, Wl2, bl2, Wl3, bl3):
    src = edge_index[0]
    dstv = edge_index[1]
    xp = jnp.pad(x, ((0, NPAD - NN), (0, 0)))
    batch_ext = jnp.pad(batch, (0, NPAD - NN))
    m = jnp.pad(jnp.ones((NN,), jnp.float32), (0, NPAD - NN))
    bp1p = jnp.broadcast_to(bp1, (16,))
    bp2p = jnp.broadcast_to(bp2, (16,))
    bp3p = jnp.broadcast_to(bp3, (16,))

    layers = (
        (W1, b1, Wp1, bp1p, K1),
        (W2, b2, Wp2, bp2p, K2),
        (W3, b3, Wp3, bp3p, K3),
    )
    src, dstv, degp = sc_prep(src, dstv, m)
    dinv = tc_degfinish(degp).reshape(NPAD)
    H = tc_mm(xp, W1)
    Hp = sc_hprep(H, dinv)
    ones = jnp.ones((NPAD,), jnp.float32)
    reads = []
    for li, (W, b, Wp, bp, k) in enumerate(layers):
        aggp = sc_conv128(Hp, src, dstv)
        F, pp = sc_epilogue(aggp, Hp, dinv, m, b, Wp[:, 0])
        saggp = sc_sconv(pp, src, dstv)
        sm = sc_sfinish(saggp, dinv, pp, m, bp)
        mn = tc_topk(sm.reshape(NPAD // FD, FD), k).reshape(NPAD)
        if li < 2:
            src, dstv, degp = sc_prep(src, dstv, mn)
            dinv_next = tc_degfinish(degp).reshape(NPAD)
        else:
            dinv_next = ones
        Xs, sump, maxp, cntp = sc_xnext(F, sm, mn, batch_ext, dinv_next)
        reads.append(tc_readout(sump, maxp, cntp))
        m = mn
        dinv = dinv_next
        if li < 2:
            Hp = tc_mm(Xs, layers[li + 1][0])

    return tc_mlp(reads[0], reads[1], reads[2], Wl1, bl1, Wl2, bl2, Wl3, bl3)


# speedup vs baseline: 17.2413x; 17.2413x over previous
import functools

import numpy as np
import jax
import jax.numpy as jnp
from jax import lax
from jax.experimental import pallas as pl
from jax.experimental.pallas import tpu as pltpu
from jax.experimental.pallas import tpu_sc as plsc

NN = 10000
EE = 320000
FD = 128
NB = 64
NPAD = 10240
DEAD = NN
NW = 32
EPW = EE // NW
CH = 80
NCHE = EPW // CH
RPT = NPAD // NW
RCH = 80
NRCH = RPT // RCH
NSUB = 16
RPS = NPAD // NSUB
K1, K2, K3 = 5000, 2500, 1250
NEG = -1.0e30
SENT = -3.0e38

_mesh = plsc.VectorSubcoreMesh(core_axis_name="c", subcore_axis_name="s")
_SC_PARAMS = pltpu.CompilerParams(needs_layout_passes=False)
_ONEHOT = [np.eye(16, dtype=np.float32)[j] for j in range(16)]


def _wid():
    return lax.axis_index("s") * 2 + lax.axis_index("c")


def _f32(shape):
    return jax.ShapeDtypeStruct(shape, jnp.float32)


def _i32(shape):
    return jax.ShapeDtypeStruct(shape, jnp.int32)


@functools.partial(
    pl.kernel,
    out_type=(_i32((EE,)), _f32((NW, NPAD))),
    mesh=_mesh,
    compiler_params=_SC_PARAMS,
    scratch_types=[
        pltpu.VMEM((NPAD,), jnp.float32),
        pltpu.VMEM((EPW,), jnp.int32),
        pltpu.VMEM((EPW,), jnp.int32),
        pltpu.VMEM((EPW,), jnp.int32),
        pltpu.VMEM((NPAD,), jnp.float32),
    ],
)
def sc_prep(src_h, dstv_h, m_h, dstv_o, degp_o,
            m_v, src_v, dst_v, dout_v, deg_v):
    w = _wid()
    base = w * EPW
    pltpu.sync_copy(m_h, m_v)
    pltpu.sync_copy(src_h.at[pl.ds(base, EPW)], src_v)
    pltpu.sync_copy(dstv_h.at[pl.ds(base, EPW)], dst_v)

    def zero(i, _):
        deg_v[pl.ds(i * 16, 16)] = jnp.zeros((16,), jnp.float32)
        return 0

    lax.fori_loop(0, NPAD // 16, zero, 0)
    ones16 = jnp.ones((16,), jnp.float32)
    dead16 = jnp.full((16,), DEAD, jnp.int32)

    def step(i, _):
        for u in range(5):
            sl = pl.ds((i * 5 + u) * 16, 16)
            si = src_v[sl]
            di = dst_v[sl]
            ms = plsc.load_gather(m_v, [si])
            md = plsc.load_gather(m_v, [di])
            ok = (ms > 0.0) & (md > 0.0)
            dn = jnp.where(ok, di, dead16)
            dout_v[sl] = dn
            plsc.addupdate_scatter(deg_v, [dn], ones16)
        return 0

    lax.fori_loop(0, EPW // 80, step, 0)
    pltpu.sync_copy(dout_v, dstv_o.at[pl.ds(base, EPW)])
    pltpu.sync_copy(deg_v, degp_o.at[w])


@functools.partial(
    pl.kernel,
    out_type=_f32((2, NPAD, FD)),
    mesh=_mesh,
    compiler_params=_SC_PARAMS,
    scratch_types=[
        pltpu.VMEM((CH,), jnp.int32),
        pltpu.VMEM((CH,), jnp.int32),
        pltpu.VMEM((CH,), jnp.int32),
        pltpu.VMEM((CH,), jnp.int32),
        pltpu.VMEM((CH, FD), jnp.float32),
        pltpu.VMEM((CH, FD), jnp.float32),
        pltpu.VMEM_SHARED((NPAD, FD), jnp.float32),
        pltpu.SemaphoreType.DMA,
        pltpu.SemaphoreType.DMA,
        pltpu.SemaphoreType.DMA,
        pltpu.SemaphoreType.DMA,
    ],
)
def sc_conv128(hp_h, src_h, dstv_h, aggp_o, isa, ida, isb, idb, rwa, rwb,
               agg_sh, sia, sib, sga, sgb):
    c = lax.axis_index("c")
    s = lax.axis_index("s")
    w = s * 2 + c
    base = w * EPW

    def zrow(i, _):
        for v in range(FD // 16):
            rwa[i, pl.ds(v * 16, 16)] = jnp.zeros((16,), jnp.float32)
        return 0

    lax.fori_loop(0, CH, zrow, 0)
    for t in range(RPS // CH):
        pltpu.sync_copy(rwa, agg_sh.at[pl.ds(s * RPS + t * CH, CH)])
    plsc.subcore_barrier()

    def issue_idx(i, bs, bd, sem):
        off = base + i * CH
        pltpu.async_copy(src_h.at[pl.ds(off, CH)], bs, sem)
        pltpu.async_copy(dstv_h.at[pl.ds(off, CH)], bd, sem)

    def wait_idx(bs, bd, sem):
        pltpu.make_async_copy(src_h.at[pl.ds(base, CH)], bs, sem).wait()
        pltpu.make_async_copy(dstv_h.at[pl.ds(base, CH)], bd, sem).wait()

    def issue_g(bs, rw, sem):
        pltpu.async_copy(hp_h.at[bs], rw, sem)

    def wait_g(bs, rw, sem):
        pltpu.make_async_copy(hp_h.at[bs], rw, sem).wait()

    issue_idx(0, isa, ida, sia)
    wait_idx(isa, ida, sia)
    issue_g(isa, rwa, sga)
    issue_idx(1, isb, idb, sib)

    def pair(g, _):
        wait_idx(isb, idb, sib)
        wait_g(isa, rwa, sga)
        issue_g(isb, rwb, sgb)
        pltpu.sync_copy(rwa, agg_sh.at[ida], add=True)
        issue_idx(2 * g + 2, isa, ida, sia)
        wait_idx(isa, ida, sia)
        issue_g(isa, rwa, sga)
        wait_g(isb, rwb, sgb)
        pltpu.sync_copy(rwb, agg_sh.at[idb], add=True)

        @pl.when(g < NCHE // 2 - 1)
        def _():
            issue_idx(2 * g + 3, isb, idb, sib)

        return 0

    lax.fori_loop(0, NCHE // 2, pair, 0)
    wait_g(isa, rwa, sga)
    pltpu.sync_copy(rwa, agg_sh.at[ida], add=True)
    plsc.subcore_barrier()
    pltpu.sync_copy(
        agg_sh.at[pl.ds(s * RPS, RPS)], aggp_o.at[c, pl.ds(s * RPS, RPS)]
    )


@functools.partial(
    pl.kernel,
    out_type=_f32((NPAD, FD)),
    mesh=_mesh,
    compiler_params=_SC_PARAMS,
    scratch_types=[
        pltpu.VMEM((RCH, FD), jnp.float32),
        pltpu.VMEM((RPT,), jnp.float32),
    ],
)
def sc_hprep(h_h, dinv_h, hp_o, hb, db):
    w = _wid()
    n0 = w * RPT
    pltpu.sync_copy(dinv_h.at[pl.ds(n0, RPT)], db)
    for ch in range(NRCH):
        r0 = n0 + ch * RCH
        pltpu.sync_copy(h_h.at[pl.ds(r0, RCH)], hb)

        def grp(g, _):
            dvec = db[pl.ds(ch * RCH + g * 16, 16)]
            for j in range(16):
                dv = dvec[j]
                r = g * 16 + j
                for v in range(FD // 16):
                    hb[r, pl.ds(v * 16, 16)] = hb[r, pl.ds(v * 16, 16)] * dv
            return 0

        lax.fori_loop(0, RCH // 16, grp, 0)
        pltpu.sync_copy(hb, hp_o.at[pl.ds(r0, RCH)])


@functools.partial(
    pl.kernel,
    out_type=(_f32((NPAD, FD)), _f32((NPAD,))),
    mesh=_mesh,
    compiler_params=_SC_PARAMS,
    scratch_types=[
        pltpu.VMEM((RCH, FD), jnp.float32),
        pltpu.VMEM((RCH, FD), jnp.float32),
        pltpu.VMEM((RCH, FD), jnp.float32),
        pltpu.VMEM((RCH, FD), jnp.float32),
        pltpu.VMEM((RPT,), jnp.float32),
        pltpu.VMEM((RPT,), jnp.float32),
        pltpu.VMEM((FD,), jnp.float32),
        pltpu.VMEM((FD,), jnp.float32),
        pltpu.VMEM((RPT,), jnp.float32),
    ],
)
def sc_epilogue(aggp_h, h_h, dinv_h, m_h, b_h, wp_h, f_o, pp_o,
                a0, a1, hb, fb, db, mb, bv, wv, ppb):
    w = _wid()
    n0 = w * RPT
    pltpu.sync_copy(dinv_h.at[pl.ds(n0, RPT)], db)
    pltpu.sync_copy(m_h.at[pl.ds(n0, RPT)], mb)
    pltpu.sync_copy(b_h, bv)
    pltpu.sync_copy(wp_h, wv)
    for ch in range(NRCH):
        r0 = n0 + ch * RCH
        pltpu.sync_copy(aggp_h.at[0, pl.ds(r0, RCH)], a0)
        pltpu.sync_copy(aggp_h.at[1, pl.ds(r0, RCH)], a1)
        pltpu.sync_copy(h_h.at[pl.ds(r0, RCH)], hb)

        def grp(g, _):
            gb = ch * RCH + g * 16
            dvec = db[pl.ds(gb, 16)]
            mvec = mb[pl.ds(gb, 16)]
            ivec = lax.iota(jnp.int32, 16)
            ppacc = jnp.zeros((16,), jnp.float32)
            for j in range(16):
                dv = dvec[j]
                mv = mvec[j]
                r = g * 16 + j
                acc = jnp.zeros((16,), jnp.float32)
                for v in range(FD // 16):
                    sl = pl.ds(v * 16, 16)
                    val = (a0[r, sl] + a1[r, sl] + hb[r, sl]) * dv + bv[sl]
                    fv = jnp.maximum(val, 0.0) * mv
                    fb[r, sl] = fv
                    acc = acc + fv * wv[sl]
                ppacc = jnp.where(ivec == j, dv * jnp.sum(acc), ppacc)
            ppb[pl.ds(gb, 16)] = ppacc
            return 0

        lax.fori_loop(0, RCH // 16, grp, 0)
        pltpu.sync_copy(fb, f_o.at[pl.ds(r0, RCH)])
    pltpu.sync_copy(ppb, pp_o.at[pl.ds(n0, RPT)])


@functools.partial(
    pl.kernel,
    out_type=_f32((NW * NPAD,)),
    mesh=_mesh,
    compiler_params=_SC_PARAMS,
    scratch_types=[
        pltpu.VMEM((NPAD,), jnp.float32),
        pltpu.VMEM((EPW,), jnp.int32),
        pltpu.VMEM((EPW,), jnp.int32),
        pltpu.VMEM((NPAD,), jnp.float32),
    ],
)
def sc_sconv(pp_h, src_h, dstv_h, saggp_o, p_v, src_v, dst_v, agg_v):
    w = _wid()
    base = w * EPW
    pltpu.sync_copy(pp_h, p_v)
    pltpu.sync_copy(src_h.at[pl.ds(base, EPW)], src_v)
    pltpu.sync_copy(dstv_h.at[pl.ds(base, EPW)], dst_v)

    def zero(i, _):
        agg_v[pl.ds(i * 16, 16)] = jnp.zeros((16,), jnp.float32)
        return 0

    lax.fori_loop(0, NPAD // 16, zero, 0)

    def step(i, _):
        for u in range(5):
            sl = pl.ds((i * 5 + u) * 16, 16)
            si = src_v[sl]
            di = dst_v[sl]
            vals = plsc.load_gather(p_v, [si])
            plsc.addupdate_scatter(agg_v, [di], vals)
        return 0

    lax.fori_loop(0, EPW // 80, step, 0)
    pltpu.sync_copy(agg_v, saggp_o.at[pl.ds(w * NPAD, NPAD)])


@functools.partial(
    pl.kernel,
    out_type=_f32((NPAD,)),
    mesh=_mesh,
    compiler_params=_SC_PARAMS,
    scratch_types=[
        pltpu.VMEM((NW * RPT,), jnp.float32),
        pltpu.VMEM((RPT,), jnp.float32),
        pltpu.VMEM((RPT,), jnp.float32),
        pltpu.VMEM((RPT,), jnp.float32),
        pltpu.VMEM((16,), jnp.float32),
        pltpu.VMEM((RPT,), jnp.float32),
    ],
)
def sc_sfinish(saggp_h, dinv_h, pp_h, m_h, bp_h, sm_o, sg, db, pb, mb, bpv, smb):
    w = _wid()
    n0 = w * RPT
    for j in range(NW):
        pltpu.sync_copy(saggp_h.at[pl.ds(j * NPAD + n0, RPT)],
                        sg.at[pl.ds(j * RPT, RPT)])
    pltpu.sync_copy(dinv_h.at[pl.ds(n0, RPT)], db)
    pltpu.sync_copy(pp_h.at[pl.ds(n0, RPT)], pb)
    pltpu.sync_copy(m_h.at[pl.ds(n0, RPT)], mb)
    pltpu.sync_copy(bp_h, bpv)

    def grp(g, _):
        sl = pl.ds(g * 16, 16)
        acc = jnp.zeros((16,), jnp.float32)
        for j in range(NW):
            acc = acc + sg[pl.ds(j * RPT + g * 16, 16)]
        sv = db[sl] * acc + db[sl] * pb[sl] + bpv[pl.ds(0, 16)]
        smb[sl] = jnp.where(mb[sl] > 0.0, sv, jnp.full((16,), SENT, jnp.float32))
        return 0

    lax.fori_loop(0, RPT // 16, grp, 0)
    pltpu.sync_copy(smb, sm_o.at[pl.ds(n0, RPT)])


@functools.partial(
    pl.kernel,
    out_type=(
        _f32((NPAD, FD)),
        _f32((NW, NB * FD)),
        _f32((NW, NB * FD)),
        _f32((NW, NB * FD)),
    ),
    mesh=_mesh,
    compiler_params=_SC_PARAMS,
    scratch_types=[
        pltpu.VMEM((RCH, FD), jnp.float32),
        pltpu.VMEM((RCH, FD), jnp.float32),
        pltpu.VMEM((RPT,), jnp.float32),
        pltpu.VMEM((RPT,), jnp.float32),
        pltpu.VMEM((RPT,), jnp.int32),
        pltpu.VMEM((RPT,), jnp.float32),
        pltpu.VMEM((NB * FD,), jnp.float32),
        pltpu.VMEM((NB * FD,), jnp.float32),
        pltpu.VMEM((NB * FD,), jnp.float32),
    ],
)
def sc_xnext(f_h, sm_h, mn_h, batch_h, dn_h, xn_o, sump_o, maxp_o, cntp_o,
             fb, xb, smb, mnb, bb, dnb, sl_, ml_, cl_):
    w = _wid()
    n0 = w * RPT
    pltpu.sync_copy(sm_h.at[pl.ds(n0, RPT)], smb)
    pltpu.sync_copy(mn_h.at[pl.ds(n0, RPT)], mnb)
    pltpu.sync_copy(batch_h.at[pl.ds(n0, RPT)], bb)
    pltpu.sync_copy(dn_h.at[pl.ds(n0, RPT)], dnb)

    def zero(i, _):
        sl_[pl.ds(i * 16, 16)] = jnp.zeros((16,), jnp.float32)
        cl_[pl.ds(i * 16, 16)] = jnp.zeros((16,), jnp.float32)
        ml_[pl.ds(i * 16, 16)] = jnp.full((16,), NEG, jnp.float32)
        return 0

    lax.fori_loop(0, NB * FD // 16, zero, 0)
    for ch in range(NRCH):
        r0 = n0 + ch * RCH
        pltpu.sync_copy(f_h.at[pl.ds(r0, RCH)], fb)

        def grp(g, _):
            gb = ch * RCH + g * 16
            svec = smb[pl.ds(gb, 16)]
            mnvec = mnb[pl.ds(gb, 16)]
            bvec = bb[pl.ds(gb, 16)]
            dnvec = dnb[pl.ds(gb, 16)]
            e = jnp.exp(-2.0 * jnp.abs(svec))
            tco = jnp.sign(svec) * (1.0 - e) / (1.0 + e) * mnvec
            sentv = (mnvec - 1.0) * 1.0e30
            for j in range(16):
                coef = tco[j]
                sent = sentv[j]
                mnr = mnvec[j]
                seg = bvec[j]
                dnr = dnvec[j]
                r = g * 16 + j
                for v in range(FD // 16):
                    sl = pl.ds(v * 16, 16)
                    xv = fb[r, sl] * coef
                    xb[r, sl] = xv * dnr
                    off = pl.ds(seg * FD + v * 16, 16)
                    sl_[off] = sl_[off] + xv
                    ml_[off] = jnp.maximum(ml_[off], xv + sent)
                    cl_[off] = cl_[off] + mnr
            return 0

        lax.fori_loop(0, RCH // 16, grp, 0)
        pltpu.sync_copy(xb, xn_o.at[pl.ds(r0, RCH)])
    pltpu.sync_copy(sl_, sump_o.at[w])
    pltpu.sync_copy(ml_, maxp_o.at[w])
    pltpu.sync_copy(cl_, cntp_o.at[w])


def _tc_mm_body(x_ref, w_ref, o_ref):
    o_ref[...] = jnp.dot(x_ref[...], w_ref[...], preferred_element_type=jnp.float32)


def tc_mm(x, w):
    bm = 1024
    return pl.pallas_call(
        _tc_mm_body,
        grid=(NPAD // bm,),
        in_specs=[
            pl.BlockSpec((bm, FD), lambda i: (i, 0)),
            pl.BlockSpec((FD, FD), lambda i: (0, 0)),
        ],
        out_specs=pl.BlockSpec((bm, FD), lambda i: (i, 0)),
        out_shape=_f32((NPAD, FD)),
    )(x, w)


def _tc_degfinish_body(degp_ref, o_ref):
    deg = jnp.sum(degp_ref[...], axis=0) + 1.0
    o_ref[...] = lax.rsqrt(deg)


def tc_degfinish(degp):
    return pl.pallas_call(
        _tc_degfinish_body,
        out_shape=_f32((NPAD // FD, FD)),
    )(degp.reshape(NW, NPAD // FD, FD))


def _tc_topk_body(k, sm_ref, o_ref):
    sm = sm_ref[...]
    ub = lax.bitcast_convert_type(sm, jnp.uint32)
    sign = (ub >> 31).astype(jnp.uint32)
    u = jnp.where(sign > 0, ~ub, ub | jnp.uint32(0x80000000))
    kf = jnp.float32(k)

    def cnt_ge(t):
        return jnp.sum((u >= t).astype(jnp.float32))

    def vbit(b, t):
        cand = t | (jnp.uint32(1) << (jnp.uint32(31) - b.astype(jnp.uint32)))
        return jnp.where(cnt_ge(cand) >= kf, cand, t)

    t = lax.fori_loop(0, 32, vbit, jnp.uint32(0))
    c_gt = jnp.sum((u > t).astype(jnp.float32))
    r = kf - c_gt
    rows = lax.broadcasted_iota(jnp.int32, sm.shape, 0)
    lanes = lax.broadcasted_iota(jnp.int32, sm.shape, 1)
    idx = rows * FD + lanes
    eq = u == t

    def cnt_le(j):
        return jnp.sum((eq & (idx <= j)).astype(jnp.float32))

    def jbit(b, j):
        cand = j & ~(jnp.int32(1) << (jnp.int32(13) - b))
        return jnp.where(cnt_le(cand) >= r, cand, j)

    j = lax.fori_loop(0, 14, jbit, jnp.int32(16383))
    o_ref[...] = ((u > t) | (eq & (idx <= j))).astype(jnp.float32)


def tc_topk(sm2, k):
    return pl.pallas_call(
        functools.partial(_tc_topk_body, k),
        out_shape=_f32((NPAD // FD, FD)),
    )(sm2)


def _tc_readout_body(sump_ref, maxp_ref, cntp_ref, o_ref):
    s = jnp.sum(sump_ref[...], axis=0)
    c = jnp.sum(cntp_ref[...], axis=0)
    mx = jnp.max(maxp_ref[...], axis=0)
    mx = jnp.where(mx < -1.0e29, 0.0, mx)
    mean = s / jnp.maximum(c, 1.0)
    o_ref[:, :FD] = mx
    o_ref[:, FD:] = mean


def tc_readout(sump, maxp, cntp):
    return pl.pallas_call(
        _tc_readout_body,
        out_shape=_f32((NB, 2 * FD)),
    )(sump.reshape(NW, NB, FD), maxp.reshape(NW, NB, FD), cntp.reshape(NW, NB, FD))


def _tc_mlp_body(x1_ref, x2_ref, x3_ref, w1_ref, b1_ref, w2_ref, b2_ref,
                 w3_ref, b3_ref, o_ref):
    z = x1_ref[...] + x2_ref[...] + x3_ref[...]
    z = jnp.maximum(jnp.dot(z, w1_ref[...], preferred_element_type=jnp.float32)
                    + b1_ref[...], 0.0)
    z = jnp.maximum(jnp.dot(z, w2_ref[...], preferred_element_type=jnp.float32)
                    + b2_ref[...], 0.0)
    lg = jnp.dot(z, w3_ref[...], preferred_element_type=jnp.float32) + b3_ref[...]
    mx = jnp.max(lg, axis=1, keepdims=True)
    ls = jnp.log(jnp.sum(jnp.exp(lg - mx), axis=1, keepdims=True)) + mx
    o_ref[...] = lg - ls


def tc_mlp(x1, x2, x3, w1, b1, w2, b2, w3, b3):
    return pl.pallas_call(
        _tc_mlp_body,
        out_shape=_f32((NB, 10)),
    )(x1, x2, x3, w1, b1.reshape(1, -1), w2, b2.reshape(1, -1),
      w3, b3.reshape(1, -1))


def kernel(x, edge_index, batch, epoch, W1, b1, Wp1, bp1, W2, b2, Wp2, bp2,
           W3, b3, Wp3, bp3, Wl1, bl1, Wl2, bl2, Wl3, bl3):
    src = edge_index[0]
    dstv = edge_index[1]
    xp = jnp.pad(x, ((0, NPAD - NN), (0, 0)))
    batch_ext = jnp.pad(batch, (0, NPAD - NN))
    m = jnp.pad(jnp.ones((NN,), jnp.float32), (0, NPAD - NN))
    bp1p = jnp.broadcast_to(bp1, (16,))
    bp2p = jnp.broadcast_to(bp2, (16,))
    bp3p = jnp.broadcast_to(bp3, (16,))

    layers = (
        (W1, b1, Wp1, bp1p, K1),
        (W2, b2, Wp2, bp2p, K2),
        (W3, b3, Wp3, bp3p, K3),
    )
    dstv, degp = sc_prep(src, dstv, m)
    dinv = tc_degfinish(degp).reshape(NPAD)
    H = tc_mm(xp, W1)
    Hp = sc_hprep(H, dinv)
    ones = jnp.ones((NPAD,), jnp.float32)
    reads = []
    for li, (W, b, Wp, bp, k) in enumerate(layers):
        aggp = sc_conv128(Hp, src, dstv)
        F, pp = sc_epilogue(aggp, Hp, dinv, m, b, Wp[:, 0])
        saggp = sc_sconv(pp, src, dstv)
        sm = sc_sfinish(saggp, dinv, pp, m, bp)
        mn = tc_topk(sm.reshape(NPAD // FD, FD), k).reshape(NPAD)
        if li < 2:
            dstv, degp = sc_prep(src, dstv, mn)
            dinv_next = tc_degfinish(degp).reshape(NPAD)
        else:
            dinv_next = ones
        Xs, sump, maxp, cntp = sc_xnext(F, sm, mn, batch_ext, dinv_next)
        reads.append(tc_readout(sump, maxp, cntp))
        m = mn
        dinv = dinv_next
        if li < 2:
            Hp = tc_mm(Xs, layers[li + 1][0])

    return tc_mlp(reads[0], reads[1], reads[2], Wl1, bl1, Wl2, bl2, Wl3, bl3)


# speedup vs baseline: 19.2006x; 1.1136x over previous
import functools

import numpy as np
import jax
import jax.numpy as jnp
from jax import lax
from jax.experimental import pallas as pl
from jax.experimental.pallas import tpu as pltpu
from jax.experimental.pallas import tpu_sc as plsc

NN = 10000
EE = 320000
FD = 128
NB = 64
NPAD = 10240
DEAD = NN
NW = 32
EPW = EE // NW
CH = 80
NCHE = EPW // CH
RPT = NPAD // NW
RCH = 80
NRCH = RPT // RCH
NSUB = 16
RPS = NPAD // NSUB
K1, K2, K3 = 5000, 2500, 1250
NEG = -1.0e30
SENT = -3.0e38

_mesh = plsc.VectorSubcoreMesh(core_axis_name="c", subcore_axis_name="s")
_SC_PARAMS = pltpu.CompilerParams(needs_layout_passes=False)
_ONEHOT = [np.eye(16, dtype=np.float32)[j] for j in range(16)]


def _wid():
    return lax.axis_index("s") * 2 + lax.axis_index("c")


def _f32(shape):
    return jax.ShapeDtypeStruct(shape, jnp.float32)


def _i32(shape):
    return jax.ShapeDtypeStruct(shape, jnp.int32)


@functools.partial(
    pl.kernel,
    out_type=(_i32((EE + NW * CH,)), _i32((EE + NW * CH,)), _i32((NW * 16,)),
              _f32((NW, NPAD))),
    mesh=_mesh,
    compiler_params=_SC_PARAMS,
    scratch_types=[
        pltpu.VMEM((NPAD,), jnp.float32),
        pltpu.VMEM((EPW + CH,), jnp.int32),
        pltpu.VMEM((EPW + CH,), jnp.int32),
        pltpu.VMEM((EPW + CH,), jnp.int32),
        pltpu.VMEM((EPW + CH,), jnp.int32),
        pltpu.VMEM((NPAD,), jnp.float32),
        pltpu.VMEM((16,), jnp.int32),
    ],
)
def sc_prep(src_h, dstv_h, cnt_h, m_h, csrc_o, cdst_o, cnt_o, degp_o,
            m_v, src_v, dst_v, cs_v, cd_v, deg_v, cnt_v):
    w = _wid()
    base = w * (EPW + CH)
    pltpu.sync_copy(m_h, m_v)
    pltpu.sync_copy(cnt_h.at[pl.ds(w * 16, 16)], cnt_v)
    n_in = cnt_v[pl.ds(0, 16)][0]
    pltpu.sync_copy(src_h.at[pl.ds(base, EPW + CH)], src_v)
    pltpu.sync_copy(dstv_h.at[pl.ds(base, EPW + CH)], dst_v)

    def zero(i, _):
        deg_v[pl.ds(i * 16, 16)] = jnp.zeros((16,), jnp.float32)
        return 0

    lax.fori_loop(0, NPAD // 16, zero, 0)
    ones16 = jnp.ones((16,), jnp.float32)
    dead16 = jnp.full((16,), DEAD, jnp.int32)
    iota16 = lax.iota(jnp.int32, 16)

    def step(i, cnt):
        for u in range(5):
            sl = pl.ds((i * 5 + u) * 16, 16)
            si = src_v[sl]
            di = dst_v[sl]
            ms = plsc.load_gather(m_v, [si])
            md = plsc.load_gather(m_v, [di])
            ok = (ms > 0.0) & (md > 0.0)
            dn = jnp.where(ok, di, dead16)
            plsc.addupdate_scatter(deg_v, [dn], ones16)
            csum = plsc.cumsum(ok.astype(jnp.int32))
            pos = cnt + csum - 1
            plsc.store_scatter(cs_v, [pos], si, mask=ok)
            plsc.store_scatter(cd_v, [pos], di, mask=ok)
            cnt = cnt + csum[15]
        return cnt

    def blk(i, cnt):
        return lax.cond(i * CH < n_in, lambda: step(i, cnt), lambda: cnt)

    cnt = lax.fori_loop(0, NCHE, blk, jnp.int32(0))
    for t in range(CH // 16):
        pos = cnt + iota16 + 16 * t
        plsc.store_scatter(cs_v, [pos], jnp.zeros((16,), jnp.int32))
        plsc.store_scatter(cd_v, [pos], dead16)
    cnt_use = ((cnt + CH) // CH) * CH
    cnt_v[pl.ds(0, 16)] = jnp.broadcast_to(cnt_use, (16,))
    pltpu.sync_copy(cs_v, csrc_o.at[pl.ds(base, EPW + CH)])
    pltpu.sync_copy(cd_v, cdst_o.at[pl.ds(base, EPW + CH)])
    pltpu.sync_copy(cnt_v, cnt_o.at[pl.ds(w * 16, 16)])
    pltpu.sync_copy(deg_v, degp_o.at[w])


@functools.partial(
    pl.kernel,
    out_type=_f32((2, NPAD, FD)),
    mesh=_mesh,
    compiler_params=_SC_PARAMS,
    scratch_types=[
        pltpu.VMEM((CH,), jnp.int32),
        pltpu.VMEM((CH,), jnp.int32),
        pltpu.VMEM((CH,), jnp.int32),
        pltpu.VMEM((CH,), jnp.int32),
        pltpu.VMEM((CH, FD), jnp.float32),
        pltpu.VMEM((CH, FD), jnp.float32),
        pltpu.VMEM_SHARED((NPAD, FD), jnp.float32),
        pltpu.VMEM((16,), jnp.int32),
        pltpu.SemaphoreType.DMA,
        pltpu.SemaphoreType.DMA,
        pltpu.SemaphoreType.DMA,
        pltpu.SemaphoreType.DMA,
    ],
)
def sc_conv128(hp_h, src_h, dstv_h, cnt_h, aggp_o, isa, ida, isb, idb, rwa, rwb,
               agg_sh, cnt_v, sia, sib, sga, sgb):
    c = lax.axis_index("c")
    s = lax.axis_index("s")
    w = s * 2 + c
    base = w * (EPW + CH)
    pltpu.sync_copy(cnt_h.at[pl.ds(w * 16, 16)], cnt_v)
    nc = cnt_v[pl.ds(0, 16)][0]

    def zrow(i, _):
        for v in range(FD // 16):
            rwa[i, pl.ds(v * 16, 16)] = jnp.zeros((16,), jnp.float32)
        return 0

    lax.fori_loop(0, CH, zrow, 0)
    for t in range(RPS // CH):
        pltpu.sync_copy(rwa, agg_sh.at[pl.ds(s * RPS + t * CH, CH)])
    plsc.subcore_barrier()

    def issue_idx(i, bs, bd, sem):
        off = base + i * CH
        pltpu.async_copy(src_h.at[pl.ds(off, CH)], bs, sem)
        pltpu.async_copy(dstv_h.at[pl.ds(off, CH)], bd, sem)

    def wait_idx(bs, bd, sem):
        pltpu.make_async_copy(src_h.at[pl.ds(base, CH)], bs, sem).wait()
        pltpu.make_async_copy(dstv_h.at[pl.ds(base, CH)], bd, sem).wait()

    def issue_g(bs, rw, sem):
        pltpu.async_copy(hp_h.at[bs], rw, sem)

    def wait_g(bs, rw, sem):
        pltpu.make_async_copy(hp_h.at[bs], rw, sem).wait()

    issue_idx(0, isa, ida, sia)
    wait_idx(isa, ida, sia)
    issue_g(isa, rwa, sga)

    @pl.when(CH < nc)
    def _():
        issue_idx(1, isb, idb, sib)

    def pair(g, _):
        c0 = (2 * g) * CH
        c1 = (2 * g + 1) * CH
        c2 = (2 * g + 2) * CH
        c3 = (2 * g + 3) * CH

        @pl.when(c1 < nc)
        def _():
            wait_idx(isb, idb, sib)

        @pl.when(c0 < nc)
        def _():
            wait_g(isa, rwa, sga)

        @pl.when(c1 < nc)
        def _():
            issue_g(isb, rwb, sgb)

        @pl.when(c0 < nc)
        def _():
            pltpu.sync_copy(rwa, agg_sh.at[ida], add=True)

        @pl.when(c2 < nc)
        def _():
            issue_idx(2 * g + 2, isa, ida, sia)
            wait_idx(isa, ida, sia)
            issue_g(isa, rwa, sga)

        @pl.when(c1 < nc)
        def _():
            wait_g(isb, rwb, sgb)
            pltpu.sync_copy(rwb, agg_sh.at[idb], add=True)

        @pl.when(c3 < nc)
        def _():
            issue_idx(2 * g + 3, isb, idb, sib)

        return 0

    lax.fori_loop(0, (NCHE + 1) // 2 + 1, pair, 0)
    plsc.subcore_barrier()
    pltpu.sync_copy(
        agg_sh.at[pl.ds(s * RPS, RPS)], aggp_o.at[c, pl.ds(s * RPS, RPS)]
    )


@functools.partial(
    pl.kernel,
    out_type=_f32((NPAD, FD)),
    mesh=_mesh,
    compiler_params=_SC_PARAMS,
    scratch_types=[
        pltpu.VMEM((RCH, FD), jnp.float32),
        pltpu.VMEM((RPT,), jnp.float32),
    ],
)
def sc_hprep(h_h, dinv_h, hp_o, hb, db):
    w = _wid()
    n0 = w * RPT
    pltpu.sync_copy(dinv_h.at[pl.ds(n0, RPT)], db)
    for ch in range(NRCH):
        r0 = n0 + ch * RCH
        pltpu.sync_copy(h_h.at[pl.ds(r0, RCH)], hb)

        def grp(g, _):
            dvec = db[pl.ds(ch * RCH + g * 16, 16)]
            for j in range(16):
                dv = dvec[j]
                r = g * 16 + j
                for v in range(FD // 16):
                    hb[r, pl.ds(v * 16, 16)] = hb[r, pl.ds(v * 16, 16)] * dv
            return 0

        lax.fori_loop(0, RCH // 16, grp, 0)
        pltpu.sync_copy(hb, hp_o.at[pl.ds(r0, RCH)])


@functools.partial(
    pl.kernel,
    out_type=(_f32((NPAD, FD)), _f32((NPAD,))),
    mesh=_mesh,
    compiler_params=_SC_PARAMS,
    scratch_types=[
        pltpu.VMEM((RCH, FD), jnp.float32),
        pltpu.VMEM((RCH, FD), jnp.float32),
        pltpu.VMEM((RCH, FD), jnp.float32),
        pltpu.VMEM((RCH, FD), jnp.float32),
        pltpu.VMEM((RPT,), jnp.float32),
        pltpu.VMEM((RPT,), jnp.float32),
        pltpu.VMEM((FD,), jnp.float32),
        pltpu.VMEM((FD,), jnp.float32),
        pltpu.VMEM((RPT,), jnp.float32),
    ],
)
def sc_epilogue(aggp_h, h_h, dinv_h, m_h, b_h, wp_h, f_o, pp_o,
                a0, a1, hb, fb, db, mb, bv, wv, ppb):
    w = _wid()
    n0 = w * RPT
    pltpu.sync_copy(dinv_h.at[pl.ds(n0, RPT)], db)
    pltpu.sync_copy(m_h.at[pl.ds(n0, RPT)], mb)
    pltpu.sync_copy(b_h, bv)
    pltpu.sync_copy(wp_h, wv)
    for ch in range(NRCH):
        r0 = n0 + ch * RCH
        pltpu.sync_copy(aggp_h.at[0, pl.ds(r0, RCH)], a0)
        pltpu.sync_copy(aggp_h.at[1, pl.ds(r0, RCH)], a1)
        pltpu.sync_copy(h_h.at[pl.ds(r0, RCH)], hb)

        def grp(g, _):
            gb = ch * RCH + g * 16
            dvec = db[pl.ds(gb, 16)]
            mvec = mb[pl.ds(gb, 16)]
            ivec = lax.iota(jnp.int32, 16)
            ppacc = jnp.zeros((16,), jnp.float32)
            for j in range(16):
                dv = dvec[j]
                mv = mvec[j]
                r = g * 16 + j
                acc = jnp.zeros((16,), jnp.float32)
                for v in range(FD // 16):
                    sl = pl.ds(v * 16, 16)
                    val = (a0[r, sl] + a1[r, sl] + hb[r, sl]) * dv + bv[sl]
                    fv = jnp.maximum(val, 0.0) * mv
                    fb[r, sl] = fv
                    acc = acc + fv * wv[sl]
                ppacc = jnp.where(ivec == j, dv * jnp.sum(acc), ppacc)
            ppb[pl.ds(gb, 16)] = ppacc
            return 0

        lax.fori_loop(0, RCH // 16, grp, 0)
        pltpu.sync_copy(fb, f_o.at[pl.ds(r0, RCH)])
    pltpu.sync_copy(ppb, pp_o.at[pl.ds(n0, RPT)])


@functools.partial(
    pl.kernel,
    out_type=_f32((NW * NPAD,)),
    mesh=_mesh,
    compiler_params=_SC_PARAMS,
    scratch_types=[
        pltpu.VMEM((NPAD,), jnp.float32),
        pltpu.VMEM((EPW + CH,), jnp.int32),
        pltpu.VMEM((EPW + CH,), jnp.int32),
        pltpu.VMEM((NPAD,), jnp.float32),
        pltpu.VMEM((16,), jnp.int32),
    ],
)
def sc_sconv(pp_h, src_h, dstv_h, cnt_h, saggp_o, p_v, src_v, dst_v, agg_v, cnt_v):
    w = _wid()
    base = w * (EPW + CH)
    pltpu.sync_copy(pp_h, p_v)
    pltpu.sync_copy(cnt_h.at[pl.ds(w * 16, 16)], cnt_v)
    nc = cnt_v[pl.ds(0, 16)][0]
    pltpu.sync_copy(src_h.at[pl.ds(base, EPW + CH)], src_v)
    pltpu.sync_copy(dstv_h.at[pl.ds(base, EPW + CH)], dst_v)

    def zero(i, _):
        agg_v[pl.ds(i * 16, 16)] = jnp.zeros((16,), jnp.float32)
        return 0

    lax.fori_loop(0, NPAD // 16, zero, 0)

    def step(i):
        for u in range(5):
            sl = pl.ds((i * 5 + u) * 16, 16)
            si = src_v[sl]
            di = dst_v[sl]
            vals = plsc.load_gather(p_v, [si])
            plsc.addupdate_scatter(agg_v, [di], vals)

    def blk(i, _):
        @pl.when(i * CH < nc)
        def _():
            step(i)

        return 0

    lax.fori_loop(0, (EPW + CH) // CH, blk, 0)
    pltpu.sync_copy(agg_v, saggp_o.at[pl.ds(w * NPAD, NPAD)])


@functools.partial(
    pl.kernel,
    out_type=_f32((NPAD,)),
    mesh=_mesh,
    compiler_params=_SC_PARAMS,
    scratch_types=[
        pltpu.VMEM((NW * RPT,), jnp.float32),
        pltpu.VMEM((RPT,), jnp.float32),
        pltpu.VMEM((RPT,), jnp.float32),
        pltpu.VMEM((RPT,), jnp.float32),
        pltpu.VMEM((16,), jnp.float32),
        pltpu.VMEM((RPT,), jnp.float32),
    ],
)
def sc_sfinish(saggp_h, dinv_h, pp_h, m_h, bp_h, sm_o, sg, db, pb, mb, bpv, smb):
    w = _wid()
    n0 = w * RPT
    for j in range(NW):
        pltpu.sync_copy(saggp_h.at[pl.ds(j * NPAD + n0, RPT)],
                        sg.at[pl.ds(j * RPT, RPT)])
    pltpu.sync_copy(dinv_h.at[pl.ds(n0, RPT)], db)
    pltpu.sync_copy(pp_h.at[pl.ds(n0, RPT)], pb)
    pltpu.sync_copy(m_h.at[pl.ds(n0, RPT)], mb)
    pltpu.sync_copy(bp_h, bpv)

    def grp(g, _):
        sl = pl.ds(g * 16, 16)
        acc = jnp.zeros((16,), jnp.float32)
        for j in range(NW):
            acc = acc + sg[pl.ds(j * RPT + g * 16, 16)]
        sv = db[sl] * acc + db[sl] * pb[sl] + bpv[pl.ds(0, 16)]
        smb[sl] = jnp.where(mb[sl] > 0.0, sv, jnp.full((16,), SENT, jnp.float32))
        return 0

    lax.fori_loop(0, RPT // 16, grp, 0)
    pltpu.sync_copy(smb, sm_o.at[pl.ds(n0, RPT)])


@functools.partial(
    pl.kernel,
    out_type=(
        _f32((NPAD, FD)),
        _f32((NW, NB * FD)),
        _f32((NW, NB * FD)),
        _f32((NW, NB * FD)),
    ),
    mesh=_mesh,
    compiler_params=_SC_PARAMS,
    scratch_types=[
        pltpu.VMEM((RCH, FD), jnp.float32),
        pltpu.VMEM((RCH, FD), jnp.float32),
        pltpu.VMEM((RPT,), jnp.float32),
        pltpu.VMEM((RPT,), jnp.float32),
        pltpu.VMEM((RPT,), jnp.int32),
        pltpu.VMEM((RPT,), jnp.float32),
        pltpu.VMEM((NB * FD,), jnp.float32),
        pltpu.VMEM((NB * FD,), jnp.float32),
        pltpu.VMEM((NB * FD,), jnp.float32),
    ],
)
def sc_xnext(f_h, sm_h, mn_h, batch_h, dn_h, xn_o, sump_o, maxp_o, cntp_o,
             fb, xb, smb, mnb, bb, dnb, sl_, ml_, cl_):
    w = _wid()
    n0 = w * RPT
    pltpu.sync_copy(sm_h.at[pl.ds(n0, RPT)], smb)
    pltpu.sync_copy(mn_h.at[pl.ds(n0, RPT)], mnb)
    pltpu.sync_copy(batch_h.at[pl.ds(n0, RPT)], bb)
    pltpu.sync_copy(dn_h.at[pl.ds(n0, RPT)], dnb)

    def zero(i, _):
        sl_[pl.ds(i * 16, 16)] = jnp.zeros((16,), jnp.float32)
        cl_[pl.ds(i * 16, 16)] = jnp.zeros((16,), jnp.float32)
        ml_[pl.ds(i * 16, 16)] = jnp.full((16,), NEG, jnp.float32)
        return 0

    lax.fori_loop(0, NB * FD // 16, zero, 0)
    for ch in range(NRCH):
        r0 = n0 + ch * RCH
        pltpu.sync_copy(f_h.at[pl.ds(r0, RCH)], fb)

        def grp(g, _):
            gb = ch * RCH + g * 16
            svec = smb[pl.ds(gb, 16)]
            mnvec = mnb[pl.ds(gb, 16)]
            bvec = bb[pl.ds(gb, 16)]
            dnvec = dnb[pl.ds(gb, 16)]
            e = jnp.exp(-2.0 * jnp.abs(svec))
            tco = jnp.sign(svec) * (1.0 - e) / (1.0 + e) * mnvec
            sentv = (mnvec - 1.0) * 1.0e30
            for j in range(16):
                coef = tco[j]
                sent = sentv[j]
                mnr = mnvec[j]
                seg = bvec[j]
                dnr = dnvec[j]
                r = g * 16 + j
                for v in range(FD // 16):
                    sl = pl.ds(v * 16, 16)
                    xv = fb[r, sl] * coef
                    xb[r, sl] = xv * dnr
                    off = pl.ds(seg * FD + v * 16, 16)
                    sl_[off] = sl_[off] + xv
                    ml_[off] = jnp.maximum(ml_[off], xv + sent)
                    cl_[off] = cl_[off] + mnr
            return 0

        lax.fori_loop(0, RCH // 16, grp, 0)
        pltpu.sync_copy(xb, xn_o.at[pl.ds(r0, RCH)])
    pltpu.sync_copy(sl_, sump_o.at[w])
    pltpu.sync_copy(ml_, maxp_o.at[w])
    pltpu.sync_copy(cl_, cntp_o.at[w])


def _tc_mm_body(x_ref, w_ref, o_ref):
    o_ref[...] = jnp.dot(x_ref[...], w_ref[...], preferred_element_type=jnp.float32)


def tc_mm(x, w):
    bm = 1024
    return pl.pallas_call(
        _tc_mm_body,
        grid=(NPAD // bm,),
        in_specs=[
            pl.BlockSpec((bm, FD), lambda i: (i, 0)),
            pl.BlockSpec((FD, FD), lambda i: (0, 0)),
        ],
        out_specs=pl.BlockSpec((bm, FD), lambda i: (i, 0)),
        out_shape=_f32((NPAD, FD)),
    )(x, w)


def _tc_degfinish_body(degp_ref, o_ref):
    deg = jnp.sum(degp_ref[...], axis=0) + 1.0
    o_ref[...] = lax.rsqrt(deg)


def tc_degfinish(degp):
    return pl.pallas_call(
        _tc_degfinish_body,
        out_shape=_f32((NPAD // FD, FD)),
    )(degp.reshape(NW, NPAD // FD, FD))


def _tc_topk_body(k, sm_ref, o_ref):
    sm = sm_ref[...]
    ub = lax.bitcast_convert_type(sm, jnp.uint32)
    sign = (ub >> 31).astype(jnp.uint32)
    u = jnp.where(sign > 0, ~ub, ub | jnp.uint32(0x80000000))
    kf = jnp.float32(k)

    def cnt_ge(t):
        return jnp.sum((u >= t).astype(jnp.float32))

    def vbit(b, t):
        cand = t | (jnp.uint32(1) << (jnp.uint32(31) - b.astype(jnp.uint32)))
        return jnp.where(cnt_ge(cand) >= kf, cand, t)

    t = lax.fori_loop(0, 32, vbit, jnp.uint32(0))
    c_gt = jnp.sum((u > t).astype(jnp.float32))
    r = kf - c_gt
    rows = lax.broadcasted_iota(jnp.int32, sm.shape, 0)
    lanes = lax.broadcasted_iota(jnp.int32, sm.shape, 1)
    idx = rows * FD + lanes
    eq = u == t

    def cnt_le(j):
        return jnp.sum((eq & (idx <= j)).astype(jnp.float32))

    def jbit(b, j):
        cand = j & ~(jnp.int32(1) << (jnp.int32(13) - b))
        return jnp.where(cnt_le(cand) >= r, cand, j)

    j = lax.fori_loop(0, 14, jbit, jnp.int32(16383))
    o_ref[...] = ((u > t) | (eq & (idx <= j))).astype(jnp.float32)


def tc_topk(sm2, k):
    return pl.pallas_call(
        functools.partial(_tc_topk_body, k),
        out_shape=_f32((NPAD // FD, FD)),
    )(sm2)


def _tc_readout_body(sump_ref, maxp_ref, cntp_ref, o_ref):
    s = jnp.sum(sump_ref[...], axis=0)
    c = jnp.sum(cntp_ref[...], axis=0)
    mx = jnp.max(maxp_ref[...], axis=0)
    mx = jnp.where(mx < -1.0e29, 0.0, mx)
    mean = s / jnp.maximum(c, 1.0)
    o_ref[:, :FD] = mx
    o_ref[:, FD:] = mean


def tc_readout(sump, maxp, cntp):
    return pl.pallas_call(
        _tc_readout_body,
        out_shape=_f32((NB, 2 * FD)),
    )(sump.reshape(NW, NB, FD), maxp.reshape(NW, NB, FD), cntp.reshape(NW, NB, FD))


def _tc_mlp_body(x1_ref, x2_ref, x3_ref, w1_ref, b1_ref, w2_ref, b2_ref,
                 w3_ref, b3_ref, o_ref):
    z = x1_ref[...] + x2_ref[...] + x3_ref[...]
    z = jnp.maximum(jnp.dot(z, w1_ref[...], preferred_element_type=jnp.float32)
                    + b1_ref[...], 0.0)
    z = jnp.maximum(jnp.dot(z, w2_ref[...], preferred_element_type=jnp.float32)
                    + b2_ref[...], 0.0)
    lg = jnp.dot(z, w3_ref[...], preferred_element_type=jnp.float32) + b3_ref[...]
    mx = jnp.max(lg, axis=1, keepdims=True)
    ls = jnp.log(jnp.sum(jnp.exp(lg - mx), axis=1, keepdims=True)) + mx
    o_ref[...] = lg - ls


def tc_mlp(x1, x2, x3, w1, b1, w2, b2, w3, b3):
    return pl.pallas_call(
        _tc_mlp_body,
        out_shape=_f32((NB, 10)),
    )(x1, x2, x3, w1, b1.reshape(1, -1), w2, b2.reshape(1, -1),
      w3, b3.reshape(1, -1))


def kernel(x, edge_index, batch, epoch, W1, b1, Wp1, bp1, W2, b2, Wp2, bp2,
           W3, b3, Wp3, bp3, Wl1, bl1, Wl2, bl2, Wl3, bl3):
    src = jnp.pad(edge_index[0].reshape(NW, EPW), ((0, 0), (0, CH))).reshape(-1)
    dstv = jnp.pad(edge_index[1].reshape(NW, EPW), ((0, 0), (0, CH))).reshape(-1)
    cnt16 = jnp.full((NW * 16,), EPW, jnp.int32)
    xp = jnp.pad(x, ((0, NPAD - NN), (0, 0)))
    batch_ext = jnp.pad(batch, (0, NPAD - NN))
    m = jnp.pad(jnp.ones((NN,), jnp.float32), (0, NPAD - NN))
    bp1p = jnp.broadcast_to(bp1, (16,))
    bp2p = jnp.broadcast_to(bp2, (16,))
    bp3p = jnp.broadcast_to(bp3, (16,))

    layers = (
        (W1, b1, Wp1, bp1p, K1),
        (W2, b2, Wp2, bp2p, K2),
        (W3, b3, Wp3, bp3p, K3),
    )
    src, dstv, cnt16, degp = sc_prep(src, dstv, cnt16, m)
    dinv = tc_degfinish(degp).reshape(NPAD)
    H = tc_mm(xp, W1)
    Hp = sc_hprep(H, dinv)
    ones = jnp.ones((NPAD,), jnp.float32)
    reads = []
    for li, (W, b, Wp, bp, k) in enumerate(layers):
        aggp = sc_conv128(Hp, src, dstv, cnt16)
        F, pp = sc_epilogue(aggp, Hp, dinv, m, b, Wp[:, 0])
        saggp = sc_sconv(pp, src, dstv, cnt16)
        sm = sc_sfinish(saggp, dinv, pp, m, bp)
        mn = tc_topk(sm.reshape(NPAD // FD, FD), k).reshape(NPAD)
        if li < 2:
            src, dstv, cnt16, degp = sc_prep(src, dstv, cnt16, mn)
            dinv_next = tc_degfinish(degp).reshape(NPAD)
        else:
            dinv_next = ones
        Xs, sump, maxp, cntp = sc_xnext(F, sm, mn, batch_ext, dinv_next)
        reads.append(tc_readout(sump, maxp, cntp))
        m = mn
        dinv = dinv_next
        if li < 2:
            Hp = tc_mm(Xs, layers[li + 1][0])

    return tc_mlp(reads[0], reads[1], reads[2], Wl1, bl1, Wl2, bl2, Wl3, bl3)
